# baseline (speedup 1.0000x reference)
import functools

import jax
import jax.numpy as jnp
from jax import lax
from jax.experimental import pallas as pl
from jax.experimental.pallas import tpu as pltpu
from jax.experimental.pallas import tpu_sc as plsc

N = 10000
E = 320000
D = 128

NC = 2
NS = 16
NW = NC * NS

NPAD = 10240
RPT = NPAD // NS
CHUNK = 128
NCHUNK = 80
EPT = CHUNK * NCHUNK
EPAD = NW * EPT
DH = D // 2
IBLK = 8

_MESH = plsc.VectorSubcoreMesh(core_axis_name="c", subcore_axis_name="s")


@functools.partial(
    pl.kernel,
    out_type=jax.ShapeDtypeStruct((NC * NPAD,), jnp.float32),
    mesh=_MESH,
    scratch_types=[
        pltpu.VMEM((NCHUNK, CHUNK), jnp.int32),
        pltpu.VMEM((CHUNK,), jnp.float32),
        pltpu.VMEM((RPT,), jnp.float32),
        pltpu.VMEM_SHARED((NPAD,), jnp.float32),
    ],
)
def _deg_kernel(dst_hbm, deg_out, idx_v, ones_v, line_v, acc):
    cid = lax.axis_index("c")
    sid = lax.axis_index("s")
    wid = cid * NS + sid
    z16 = jnp.zeros((16,), jnp.float32)
    for i in range(RPT // 16):
        line_v[pl.ds(i * 16, 16)] = z16
    for i in range(CHUNK // 16):
        ones_v[pl.ds(i * 16, 16)] = jnp.ones((16,), jnp.float32)
    pltpu.sync_copy(line_v, acc.at[pl.ds(sid * RPT, RPT)])
    plsc.subcore_barrier()
    pltpu.sync_copy(dst_hbm.at[wid], idx_v)

    def body(j, carry):
        pltpu.sync_copy(ones_v, acc.at[idx_v.at[j]], add=True)
        return carry

    lax.fori_loop(0, NCHUNK, body, 0)
    plsc.subcore_barrier()
    pltpu.sync_copy(acc.at[pl.ds(sid * RPT, RPT)], line_v)
    pltpu.sync_copy(line_v, deg_out.at[pl.ds(cid * NPAD + sid * RPT, RPT)])


@functools.partial(
    pl.kernel,
    out_type=jax.ShapeDtypeStruct((2, NC * NPAD, DH), jnp.float32),
    mesh=_MESH,
    compiler_params=pltpu.CompilerParams(use_tc_tiling_on_sc=False),
    scratch_types=[
        pltpu.VMEM((2 * IBLK, CHUNK), jnp.int32),
        pltpu.VMEM((2 * IBLK, CHUNK), jnp.int32),
        pltpu.VMEM((CHUNK, DH), jnp.float32),
        pltpu.VMEM((CHUNK, DH), jnp.float32),
        pltpu.VMEM_SHARED((NPAD, DH), jnp.float32),
        pltpu.VMEM_SHARED((NPAD, DH), jnp.float32),
        pltpu.SemaphoreType.DMA,
        pltpu.SemaphoreType.DMA,
        pltpu.SemaphoreType.DMA,
        pltpu.SemaphoreType.DMA,
        pltpu.SemaphoreType.DMA,
    ],
)
def _scatter_kernel(src_hbm, dst_hbm, tab_hbm, out_hbm, src_v, dst_v,
                    rows0, rows1, tabs, acch, gsem0, gsem1, ssem0, ssem1,
                    isem):
    cid = lax.axis_index("c")
    sid = lax.axis_index("s")
    wid = cid * NS + sid
    rows = (rows0, rows1)
    gsem = (gsem0, gsem1)
    ssem = (ssem0, ssem1)
    z16 = jnp.zeros((16,), jnp.float32)

    for h in range(2):
        for r in range(CHUNK):
            for c in range(DH // 16):
                rows0[r, pl.ds(c * 16, 16)] = z16
        for k in range(RPT // CHUNK):
            base = sid * RPT + k * CHUNK
            pltpu.sync_copy(rows0, acch.at[pl.ds(base, CHUNK)])
            pltpu.sync_copy(tab_hbm.at[h, pl.ds(base, CHUNK)],
                            tabs.at[pl.ds(base, CHUNK)])
        plsc.subcore_barrier()
        pltpu.sync_copy(src_hbm.at[wid, pl.ds(0, IBLK)],
                        src_v.at[pl.ds(0, IBLK)])
        pltpu.sync_copy(dst_hbm.at[wid, pl.ds(0, IBLK)],
                        dst_v.at[pl.ds(0, IBLK)])
        pltpu.async_copy(tabs.at[src_v.at[0]], rows0, gsem0)

        def body(i, carry):
            for b in range(2):
                j = 2 * i + b
                jm = j % (2 * IBLK)
                jn = (j + 1) % (2 * IBLK)
                jp = (j - 1) % (2 * IBLK)
                pltpu.make_async_copy(tabs.at[src_v.at[jm]], rows[b],
                                      gsem[b]).wait()
                if b == 0:
                    @pl.when(i > 0)
                    def _():
                        pltpu.make_async_copy(rows[1], acch.at[dst_v.at[jp]],
                                              ssem[1]).wait()

                    @pl.when(jnp.logical_and(i % (IBLK // 2) == 0,
                                             j + IBLK < NCHUNK))
                    def _():
                        nxt = pl.multiple_of(j + IBLK, IBLK)
                        off = pl.multiple_of((jm + IBLK) % (2 * IBLK), IBLK)
                        pltpu.async_copy(src_hbm.at[wid, pl.ds(nxt, IBLK)],
                                         src_v.at[pl.ds(off, IBLK)], isem)
                        pltpu.async_copy(dst_hbm.at[wid, pl.ds(nxt, IBLK)],
                                         dst_v.at[pl.ds(off, IBLK)], isem)
                    pltpu.async_copy(tabs.at[src_v.at[jn]], rows[1], gsem[1])
                else:
                    pltpu.make_async_copy(rows[0], acch.at[dst_v.at[jp]],
                                          ssem[0]).wait()

                    @pl.when(jnp.logical_and(i % (IBLK // 2) == IBLK // 2 - 1,
                                             j + 1 < NCHUNK))
                    def _():
                        nxt = pl.multiple_of(j + 1, IBLK)
                        off = pl.multiple_of((j + 1) % (2 * IBLK), IBLK)
                        pltpu.make_async_copy(
                            src_hbm.at[wid, pl.ds(nxt, IBLK)],
                            src_v.at[pl.ds(off, IBLK)], isem).wait()
                        pltpu.make_async_copy(
                            dst_hbm.at[wid, pl.ds(nxt, IBLK)],
                            dst_v.at[pl.ds(off, IBLK)], isem).wait()

                    @pl.when(i < NCHUNK // 2 - 1)
                    def _():
                        pltpu.async_copy(tabs.at[src_v.at[jn]], rows[0],
                                         gsem[0])
                pltpu.async_copy(rows[b], acch.at[dst_v.at[jm]], ssem[b],
                                 add=True)
            return carry

        lax.fori_loop(0, NCHUNK // 2, body, 0)
        pltpu.make_async_copy(
            rows1, acch.at[dst_v.at[(NCHUNK - 1) % (2 * IBLK)]], ssem1).wait()
        plsc.subcore_barrier()
        for k in range(RPT // CHUNK):
            base = sid * RPT + k * CHUNK
            pltpu.sync_copy(acch.at[pl.ds(base, CHUNK)], rows0)
            pltpu.sync_copy(rows0, out_hbm.at[h, pl.ds(cid * NPAD + base,
                                                       CHUNK)])


_BLK = 1000


def _dinv_body(degp_ref, dinv_ref):
    deg = degp_ref[0, :] + degp_ref[1, :] + 1.0
    dinv_ref[...] = lax.rsqrt(deg)[:, None]


def _mm1_body(x_ref, w_ref, dinv_ref, h1s_ref):
    dinv = dinv_ref[...]
    h = jnp.dot(x_ref[...], w_ref[...], preferred_element_type=jnp.float32)
    hs = h * dinv
    h1s_ref[0] = hs[:, :DH]
    h1s_ref[1] = hs[:, DH:]


def _mid_body(acc_ref, h1s_ref, dinv_ref, b1_ref, w2_ref, h2s_ref):
    tot = jnp.concatenate(
        [acc_ref[0, 0] + acc_ref[0, 1] + h1s_ref[0],
         acc_ref[1, 0] + acc_ref[1, 1] + h1s_ref[1]], axis=-1)
    dinv = dinv_ref[...]
    h1 = jnp.maximum(tot * dinv + b1_ref[...], 0.0)
    h2s = jnp.dot(h1, w2_ref[...], preferred_element_type=jnp.float32) * dinv
    h2s_ref[0] = h2s[:, :DH]
    h2s_ref[1] = h2s[:, DH:]


def _fin_body(acc_ref, h2s_ref, dinv_ref, b2_ref, out_ref):
    tot = jnp.concatenate(
        [acc_ref[0, 0] + acc_ref[0, 1] + h2s_ref[0],
         acc_ref[1, 0] + acc_ref[1, 1] + h2s_ref[1]], axis=-1)
    out_ref[...] = tot * dinv_ref[...] + b2_ref[...]


def kernel(x, edge_index, W1, b1, W2, b2):
    src = edge_index[0]
    dst = edge_index[1]
    pad = EPAD - E
    src_p = jnp.concatenate([src, jnp.zeros((pad,), jnp.int32)]).reshape(NW, NCHUNK, CHUNK)
    dst_p = jnp.concatenate([dst, jnp.full((pad,), N, jnp.int32)]).reshape(NW, NCHUNK, CHUNK)

    degp = _deg_kernel(dst_p).reshape(NC, NPAD)

    dinv = pl.pallas_call(
        _dinv_body,
        grid=(NPAD // 1024,),
        in_specs=[pl.BlockSpec((NC, 1024), lambda i: (0, i))],
        out_specs=pl.BlockSpec((1024, 1), lambda i: (i, 0)),
        out_shape=jax.ShapeDtypeStruct((NPAD, 1), jnp.float32),
    )(degp)

    grid = (N // _BLK,)
    h1s = pl.pallas_call(
        _mm1_body,
        grid=grid,
        in_specs=[
            pl.BlockSpec((_BLK, D), lambda i: (i, 0)),
            pl.BlockSpec((D, D), lambda i: (0, 0)),
            pl.BlockSpec((_BLK, 1), lambda i: (i, 0)),
        ],
        out_specs=pl.BlockSpec((2, _BLK, DH), lambda i: (0, i, 0)),
        out_shape=jax.ShapeDtypeStruct((2, NPAD, DH), jnp.float32),
    )(x, W1, dinv)

    acc1 = _scatter_kernel(src_p, dst_p, h1s).reshape(2, NC, NPAD, DH)

    h2s = pl.pallas_call(
        _mid_body,
        grid=grid,
        in_specs=[
            pl.BlockSpec((2, NC, _BLK, DH), lambda i: (0, 0, i, 0)),
            pl.BlockSpec((2, _BLK, DH), lambda i: (0, i, 0)),
            pl.BlockSpec((_BLK, 1), lambda i: (i, 0)),
            pl.BlockSpec((1, D), lambda i: (0, 0)),
            pl.BlockSpec((D, D), lambda i: (0, 0)),
        ],
        out_specs=pl.BlockSpec((2, _BLK, DH), lambda i: (0, i, 0)),
        out_shape=jax.ShapeDtypeStruct((2, NPAD, DH), jnp.float32),
    )(acc1, h1s, dinv, b1.reshape(1, D), W2)

    acc2 = _scatter_kernel(src_p, dst_p, h2s).reshape(2, NC, NPAD, DH)

    out = pl.pallas_call(
        _fin_body,
        grid=grid,
        in_specs=[
            pl.BlockSpec((2, NC, _BLK, DH), lambda i: (0, 0, i, 0)),
            pl.BlockSpec((2, _BLK, DH), lambda i: (0, i, 0)),
            pl.BlockSpec((_BLK, 1), lambda i: (i, 0)),
            pl.BlockSpec((1, D), lambda i: (0, 0)),
        ],
        out_specs=pl.BlockSpec((_BLK, D), lambda i: (i, 0)),
        out_shape=jax.ShapeDtypeStruct((N, D), jnp.float32),
    )(acc2, h2s, dinv, b2.reshape(1, D))

    return out


# speedup vs baseline: 1.0407x; 1.0407x over previous
import functools

import jax
import jax.numpy as jnp
from jax import lax
from jax.experimental import pallas as pl
from jax.experimental.pallas import tpu as pltpu
from jax.experimental.pallas import tpu_sc as plsc

N = 10000
E = 320000
D = 128

NC = 2
NS = 16
NW = NC * NS

NPAD = 10240
RPT = NPAD // NS
CHUNK = 128
NCHUNK = 80
EPT = CHUNK * NCHUNK
EPAD = NW * EPT
DH = D // 2
IBLK = 8

_MESH = plsc.VectorSubcoreMesh(core_axis_name="c", subcore_axis_name="s")


@functools.partial(
    pl.kernel,
    out_type=jax.ShapeDtypeStruct((NC * NPAD,), jnp.float32),
    mesh=_MESH,
    scratch_types=[
        pltpu.VMEM((NCHUNK, CHUNK), jnp.int32),
        pltpu.VMEM((CHUNK,), jnp.float32),
        pltpu.VMEM((RPT,), jnp.float32),
        pltpu.VMEM_SHARED((NPAD,), jnp.float32),
    ],
)
def _deg_kernel(dst_hbm, deg_out, idx_v, ones_v, line_v, acc):
    cid = lax.axis_index("c")
    sid = lax.axis_index("s")
    wid = cid * NS + sid
    z16 = jnp.zeros((16,), jnp.float32)
    for i in range(RPT // 16):
        line_v[pl.ds(i * 16, 16)] = z16
    for i in range(CHUNK // 16):
        ones_v[pl.ds(i * 16, 16)] = jnp.ones((16,), jnp.float32)
    pltpu.sync_copy(line_v, acc.at[pl.ds(sid * RPT, RPT)])
    plsc.subcore_barrier()
    pltpu.sync_copy(dst_hbm.at[wid], idx_v)

    def body(j, carry):
        pltpu.sync_copy(ones_v, acc.at[idx_v.at[j]], add=True)
        return carry

    lax.fori_loop(0, NCHUNK, body, 0)
    plsc.subcore_barrier()
    pltpu.sync_copy(acc.at[pl.ds(sid * RPT, RPT)], line_v)
    pltpu.sync_copy(line_v, deg_out.at[pl.ds(cid * NPAD + sid * RPT, RPT)])


@functools.partial(
    pl.kernel,
    out_type=jax.ShapeDtypeStruct((2, NC * NPAD, DH), jnp.float32),
    mesh=_MESH,
    compiler_params=pltpu.CompilerParams(use_tc_tiling_on_sc=False),
    scratch_types=[
        pltpu.VMEM((2 * IBLK, CHUNK), jnp.int32),
        pltpu.VMEM((2 * IBLK, CHUNK), jnp.int32),
        pltpu.VMEM((CHUNK, DH), jnp.float32),
        pltpu.VMEM((CHUNK, DH), jnp.float32),
        pltpu.VMEM_SHARED((NPAD, DH), jnp.float32),
        pltpu.VMEM_SHARED((NPAD, DH), jnp.float32),
        pltpu.SemaphoreType.DMA,
        pltpu.SemaphoreType.DMA,
        pltpu.SemaphoreType.DMA,
        pltpu.SemaphoreType.DMA,
        pltpu.SemaphoreType.DMA,
    ],
)
def _scatter_kernel(src_hbm, dst_hbm, tab_hbm, out_hbm, src_v, dst_v,
                    rows0, rows1, tabs, acch, gsem0, gsem1, ssem0, ssem1,
                    isem):
    cid = lax.axis_index("c")
    sid = lax.axis_index("s")
    wid = cid * NS + sid
    rows = (rows0, rows1)
    gsem = (gsem0, gsem1)
    ssem = (ssem0, ssem1)
    z16 = jnp.zeros((16,), jnp.float32)

    for h in range(2):
        for r in range(CHUNK):
            for c in range(DH // 16):
                rows0[r, pl.ds(c * 16, 16)] = z16
        for k in range(RPT // CHUNK):
            base = sid * RPT + k * CHUNK
            pltpu.async_copy(rows0, acch.at[pl.ds(base, CHUNK)], ssem0)
            pltpu.async_copy(tab_hbm.at[h, pl.ds(base, CHUNK)],
                             tabs.at[pl.ds(base, CHUNK)], ssem1)
        for k in range(RPT // CHUNK):
            base = sid * RPT + k * CHUNK
            pltpu.make_async_copy(rows0, acch.at[pl.ds(base, CHUNK)],
                                  ssem0).wait()
            pltpu.make_async_copy(tab_hbm.at[h, pl.ds(base, CHUNK)],
                                  tabs.at[pl.ds(base, CHUNK)], ssem1).wait()
        plsc.subcore_barrier()
        pltpu.sync_copy(src_hbm.at[wid, pl.ds(0, IBLK)],
                        src_v.at[pl.ds(0, IBLK)])
        pltpu.sync_copy(dst_hbm.at[wid, pl.ds(0, IBLK)],
                        dst_v.at[pl.ds(0, IBLK)])
        pltpu.async_copy(tabs.at[src_v.at[0]], rows0, gsem0)

        def body(i, carry):
            for b in range(2):
                j = 2 * i + b
                jm = j % (2 * IBLK)
                jn = (j + 1) % (2 * IBLK)
                jp = (j - 1) % (2 * IBLK)
                pltpu.make_async_copy(tabs.at[src_v.at[jm]], rows[b],
                                      gsem[b]).wait()
                if b == 0:
                    @pl.when(i > 0)
                    def _():
                        pltpu.make_async_copy(rows[1], acch.at[dst_v.at[jp]],
                                              ssem[1]).wait()

                    @pl.when(jnp.logical_and(i % (IBLK // 2) == 0,
                                             j + IBLK < NCHUNK))
                    def _():
                        nxt = pl.multiple_of(j + IBLK, IBLK)
                        off = pl.multiple_of((jm + IBLK) % (2 * IBLK), IBLK)
                        pltpu.async_copy(src_hbm.at[wid, pl.ds(nxt, IBLK)],
                                         src_v.at[pl.ds(off, IBLK)], isem)
                        pltpu.async_copy(dst_hbm.at[wid, pl.ds(nxt, IBLK)],
                                         dst_v.at[pl.ds(off, IBLK)], isem)
                    pltpu.async_copy(tabs.at[src_v.at[jn]], rows[1], gsem[1])
                else:
                    pltpu.make_async_copy(rows[0], acch.at[dst_v.at[jp]],
                                          ssem[0]).wait()

                    @pl.when(jnp.logical_and(i % (IBLK // 2) == IBLK // 2 - 1,
                                             j + 1 < NCHUNK))
                    def _():
                        nxt = pl.multiple_of(j + 1, IBLK)
                        off = pl.multiple_of((j + 1) % (2 * IBLK), IBLK)
                        pltpu.make_async_copy(
                            src_hbm.at[wid, pl.ds(nxt, IBLK)],
                            src_v.at[pl.ds(off, IBLK)], isem).wait()
                        pltpu.make_async_copy(
                            dst_hbm.at[wid, pl.ds(nxt, IBLK)],
                            dst_v.at[pl.ds(off, IBLK)], isem).wait()

                    @pl.when(i < NCHUNK // 2 - 1)
                    def _():
                        pltpu.async_copy(tabs.at[src_v.at[jn]], rows[0],
                                         gsem[0])
                pltpu.async_copy(rows[b], acch.at[dst_v.at[jm]], ssem[b],
                                 add=True)
            return carry

        lax.fori_loop(0, NCHUNK // 2, body, 0)
        pltpu.make_async_copy(
            rows1, acch.at[dst_v.at[(NCHUNK - 1) % (2 * IBLK)]], ssem1).wait()
        plsc.subcore_barrier()
        nko = RPT // CHUNK
        for k in range(nko):
            b = k % 2
            base = sid * RPT + k * CHUNK
            if k >= 2:
                pb = sid * RPT + (k - 2) * CHUNK
                pltpu.make_async_copy(
                    rows[b], out_hbm.at[h, pl.ds(cid * NPAD + pb, CHUNK)],
                    ssem[b]).wait()
            pltpu.async_copy(acch.at[pl.ds(base, CHUNK)], rows[b], gsem[b])
            pltpu.make_async_copy(acch.at[pl.ds(base, CHUNK)], rows[b],
                                  gsem[b]).wait()
            pltpu.async_copy(rows[b],
                             out_hbm.at[h, pl.ds(cid * NPAD + base, CHUNK)],
                             ssem[b])
        for k in range(nko - 2, nko):
            b = k % 2
            base = sid * RPT + k * CHUNK
            pltpu.make_async_copy(
                rows[b], out_hbm.at[h, pl.ds(cid * NPAD + base, CHUNK)],
                ssem[b]).wait()


_BLK = 1000


def _dinv_body(degp_ref, dinv_ref):
    deg = degp_ref[0, :] + degp_ref[1, :] + 1.0
    dinv_ref[...] = lax.rsqrt(deg)[:, None]


def _mm1_body(x_ref, w_ref, dinv_ref, h1s_ref):
    dinv = dinv_ref[...]
    h = jnp.dot(x_ref[...], w_ref[...], preferred_element_type=jnp.float32)
    hs = h * dinv
    h1s_ref[0] = hs[:, :DH]
    h1s_ref[1] = hs[:, DH:]


def _mid_body(acc_ref, h1s_ref, dinv_ref, b1_ref, w2_ref, h2s_ref):
    tot = jnp.concatenate(
        [acc_ref[0, 0] + acc_ref[0, 1] + h1s_ref[0],
         acc_ref[1, 0] + acc_ref[1, 1] + h1s_ref[1]], axis=-1)
    dinv = dinv_ref[...]
    h1 = jnp.maximum(tot * dinv + b1_ref[...], 0.0)
    h2s = jnp.dot(h1, w2_ref[...], preferred_element_type=jnp.float32) * dinv
    h2s_ref[0] = h2s[:, :DH]
    h2s_ref[1] = h2s[:, DH:]


def _fin_body(acc_ref, h2s_ref, dinv_ref, b2_ref, out_ref):
    tot = jnp.concatenate(
        [acc_ref[0, 0] + acc_ref[0, 1] + h2s_ref[0],
         acc_ref[1, 0] + acc_ref[1, 1] + h2s_ref[1]], axis=-1)
    out_ref[...] = tot * dinv_ref[...] + b2_ref[...]


def kernel(x, edge_index, W1, b1, W2, b2):
    src = edge_index[0]
    dst = edge_index[1]
    pad = EPAD - E
    src_p = jnp.concatenate([src, jnp.zeros((pad,), jnp.int32)]).reshape(NW, NCHUNK, CHUNK)
    dst_p = jnp.concatenate([dst, jnp.full((pad,), N, jnp.int32)]).reshape(NW, NCHUNK, CHUNK)

    degp = _deg_kernel(dst_p).reshape(NC, NPAD)

    dinv = pl.pallas_call(
        _dinv_body,
        grid=(NPAD // 1024,),
        in_specs=[pl.BlockSpec((NC, 1024), lambda i: (0, i))],
        out_specs=pl.BlockSpec((1024, 1), lambda i: (i, 0)),
        out_shape=jax.ShapeDtypeStruct((NPAD, 1), jnp.float32),
    )(degp)

    grid = (N // _BLK,)
    h1s = pl.pallas_call(
        _mm1_body,
        grid=grid,
        in_specs=[
            pl.BlockSpec((_BLK, D), lambda i: (i, 0)),
            pl.BlockSpec((D, D), lambda i: (0, 0)),
            pl.BlockSpec((_BLK, 1), lambda i: (i, 0)),
        ],
        out_specs=pl.BlockSpec((2, _BLK, DH), lambda i: (0, i, 0)),
        out_shape=jax.ShapeDtypeStruct((2, NPAD, DH), jnp.float32),
    )(x, W1, dinv)

    acc1 = _scatter_kernel(src_p, dst_p, h1s).reshape(2, NC, NPAD, DH)

    h2s = pl.pallas_call(
        _mid_body,
        grid=grid,
        in_specs=[
            pl.BlockSpec((2, NC, _BLK, DH), lambda i: (0, 0, i, 0)),
            pl.BlockSpec((2, _BLK, DH), lambda i: (0, i, 0)),
            pl.BlockSpec((_BLK, 1), lambda i: (i, 0)),
            pl.BlockSpec((1, D), lambda i: (0, 0)),
            pl.BlockSpec((D, D), lambda i: (0, 0)),
        ],
        out_specs=pl.BlockSpec((2, _BLK, DH), lambda i: (0, i, 0)),
        out_shape=jax.ShapeDtypeStruct((2, NPAD, DH), jnp.float32),
    )(acc1, h1s, dinv, b1.reshape(1, D), W2)

    acc2 = _scatter_kernel(src_p, dst_p, h2s).reshape(2, NC, NPAD, DH)

    out = pl.pallas_call(
        _fin_body,
        grid=grid,
        in_specs=[
            pl.BlockSpec((2, NC, _BLK, DH), lambda i: (0, 0, i, 0)),
            pl.BlockSpec((2, _BLK, DH), lambda i: (0, i, 0)),
            pl.BlockSpec((_BLK, 1), lambda i: (i, 0)),
            pl.BlockSpec((1, D), lambda i: (0, 0)),
        ],
        out_specs=pl.BlockSpec((_BLK, D), lambda i: (i, 0)),
        out_shape=jax.ShapeDtypeStruct((N, D), jnp.float32),
    )(acc2, h2s, dinv, b2.reshape(1, D))

    return out


# speedup vs baseline: 1.0489x; 1.0079x over previous
import functools

import jax
import jax.numpy as jnp
from jax import lax
from jax.experimental import pallas as pl
from jax.experimental.pallas import tpu as pltpu
from jax.experimental.pallas import tpu_sc as plsc

N = 10000
E = 320000
D = 128

NC = 2
NS = 16
NW = NC * NS

NPAD = 10240
RPT = NPAD // NS
CHUNK = 128
NCHUNK = 80
EPT = CHUNK * NCHUNK
EPAD = NW * EPT
DH = D // 2
IBLK = 8

_MESH = plsc.VectorSubcoreMesh(core_axis_name="c", subcore_axis_name="s")


@functools.partial(
    pl.kernel,
    out_type=jax.ShapeDtypeStruct((NC * NPAD,), jnp.float32),
    mesh=_MESH,
    scratch_types=[
        pltpu.VMEM((NCHUNK, CHUNK), jnp.int32),
        pltpu.VMEM((CHUNK,), jnp.float32),
        pltpu.VMEM((RPT,), jnp.float32),
        pltpu.VMEM_SHARED((NPAD,), jnp.float32),
        pltpu.SemaphoreType.DMA,
    ],
)
def _deg_kernel(dst_hbm, deg_out, idx_v, ones_v, line_v, acc, dsem):
    cid = lax.axis_index("c")
    sid = lax.axis_index("s")
    wid = cid * NS + sid
    z16 = jnp.zeros((16,), jnp.float32)
    for i in range(RPT // 16):
        line_v[pl.ds(i * 16, 16)] = z16
    for i in range(CHUNK // 16):
        ones_v[pl.ds(i * 16, 16)] = jnp.ones((16,), jnp.float32)
    pltpu.sync_copy(line_v, acc.at[pl.ds(sid * RPT, RPT)])
    plsc.subcore_barrier()
    pltpu.sync_copy(dst_hbm.at[wid], idx_v)

    def body(g, carry):
        for b in range(8):
            pltpu.async_copy(ones_v, acc.at[idx_v.at[8 * g + b]], dsem,
                             add=True)
        for b in range(8):
            pltpu.make_async_copy(ones_v, acc.at[idx_v.at[8 * g + b]],
                                  dsem).wait()
        return carry

    lax.fori_loop(0, NCHUNK // 8, body, 0)
    plsc.subcore_barrier()
    pltpu.sync_copy(acc.at[pl.ds(sid * RPT, RPT)], line_v)
    pltpu.sync_copy(line_v, deg_out.at[pl.ds(cid * NPAD + sid * RPT, RPT)])


@functools.partial(
    pl.kernel,
    out_type=jax.ShapeDtypeStruct((2, NC * NPAD, DH), jnp.float32),
    mesh=_MESH,
    compiler_params=pltpu.CompilerParams(use_tc_tiling_on_sc=False),
    scratch_types=[
        pltpu.VMEM((2 * IBLK, CHUNK), jnp.int32),
        pltpu.VMEM((2 * IBLK, CHUNK), jnp.int32),
        pltpu.VMEM((CHUNK, DH), jnp.float32),
        pltpu.VMEM((CHUNK, DH), jnp.float32),
        pltpu.VMEM_SHARED((NPAD, DH), jnp.float32),
        pltpu.VMEM_SHARED((NPAD, DH), jnp.float32),
        pltpu.SemaphoreType.DMA,
        pltpu.SemaphoreType.DMA,
        pltpu.SemaphoreType.DMA,
        pltpu.SemaphoreType.DMA,
        pltpu.SemaphoreType.DMA,
    ],
)
def _scatter_kernel(src_hbm, dst_hbm, tab_hbm, out_hbm, src_v, dst_v,
                    rows0, rows1, tabs, acch, gsem0, gsem1, ssem0, ssem1,
                    isem):
    cid = lax.axis_index("c")
    sid = lax.axis_index("s")
    wid = cid * NS + sid
    rows = (rows0, rows1)
    gsem = (gsem0, gsem1)
    ssem = (ssem0, ssem1)
    z16 = jnp.zeros((16,), jnp.float32)

    for h in range(2):
        for r in range(CHUNK):
            for c in range(DH // 16):
                rows0[r, pl.ds(c * 16, 16)] = z16
        for k in range(RPT // CHUNK):
            base = sid * RPT + k * CHUNK
            pltpu.async_copy(rows0, acch.at[pl.ds(base, CHUNK)], ssem0)
            pltpu.async_copy(tab_hbm.at[h, pl.ds(base, CHUNK)],
                             tabs.at[pl.ds(base, CHUNK)], ssem1)
        for k in range(RPT // CHUNK):
            base = sid * RPT + k * CHUNK
            pltpu.make_async_copy(rows0, acch.at[pl.ds(base, CHUNK)],
                                  ssem0).wait()
            pltpu.make_async_copy(tab_hbm.at[h, pl.ds(base, CHUNK)],
                                  tabs.at[pl.ds(base, CHUNK)], ssem1).wait()
        plsc.subcore_barrier()
        pltpu.sync_copy(src_hbm.at[wid, pl.ds(0, IBLK)],
                        src_v.at[pl.ds(0, IBLK)])
        pltpu.sync_copy(dst_hbm.at[wid, pl.ds(0, IBLK)],
                        dst_v.at[pl.ds(0, IBLK)])
        pltpu.async_copy(tabs.at[src_v.at[0]], rows0, gsem0)

        def body(i, carry):
            for b in range(2):
                j = 2 * i + b
                jm = j % (2 * IBLK)
                jn = (j + 1) % (2 * IBLK)
                jp = (j - 1) % (2 * IBLK)
                pltpu.make_async_copy(tabs.at[src_v.at[jm]], rows[b],
                                      gsem[b]).wait()
                if b == 0:
                    @pl.when(i > 0)
                    def _():
                        pltpu.make_async_copy(rows[1], acch.at[dst_v.at[jp]],
                                              ssem[1]).wait()

                    @pl.when(jnp.logical_and(i % (IBLK // 2) == 0,
                                             j + IBLK < NCHUNK))
                    def _():
                        nxt = pl.multiple_of(j + IBLK, IBLK)
                        off = pl.multiple_of((jm + IBLK) % (2 * IBLK), IBLK)
                        pltpu.async_copy(src_hbm.at[wid, pl.ds(nxt, IBLK)],
                                         src_v.at[pl.ds(off, IBLK)], isem)
                        pltpu.async_copy(dst_hbm.at[wid, pl.ds(nxt, IBLK)],
                                         dst_v.at[pl.ds(off, IBLK)], isem)
                    pltpu.async_copy(tabs.at[src_v.at[jn]], rows[1], gsem[1])
                else:
                    pltpu.make_async_copy(rows[0], acch.at[dst_v.at[jp]],
                                          ssem[0]).wait()

                    @pl.when(jnp.logical_and(i % (IBLK // 2) == IBLK // 2 - 1,
                                             j + 1 < NCHUNK))
                    def _():
                        nxt = pl.multiple_of(j + 1, IBLK)
                        off = pl.multiple_of((j + 1) % (2 * IBLK), IBLK)
                        pltpu.make_async_copy(
                            src_hbm.at[wid, pl.ds(nxt, IBLK)],
                            src_v.at[pl.ds(off, IBLK)], isem).wait()
                        pltpu.make_async_copy(
                            dst_hbm.at[wid, pl.ds(nxt, IBLK)],
                            dst_v.at[pl.ds(off, IBLK)], isem).wait()

                    @pl.when(i < NCHUNK // 2 - 1)
                    def _():
                        pltpu.async_copy(tabs.at[src_v.at[jn]], rows[0],
                                         gsem[0])
                pltpu.async_copy(rows[b], acch.at[dst_v.at[jm]], ssem[b],
                                 add=True)
            return carry

        lax.fori_loop(0, NCHUNK // 2, body, 0)
        pltpu.make_async_copy(
            rows1, acch.at[dst_v.at[(NCHUNK - 1) % (2 * IBLK)]], ssem1).wait()
        plsc.subcore_barrier()
        nko = RPT // CHUNK
        for k in range(nko):
            b = k % 2
            base = sid * RPT + k * CHUNK
            if k >= 2:
                pb = sid * RPT + (k - 2) * CHUNK
                pltpu.make_async_copy(
                    rows[b], out_hbm.at[h, pl.ds(cid * NPAD + pb, CHUNK)],
                    ssem[b]).wait()
            pltpu.async_copy(acch.at[pl.ds(base, CHUNK)], rows[b], gsem[b])
            pltpu.make_async_copy(acch.at[pl.ds(base, CHUNK)], rows[b],
                                  gsem[b]).wait()
            pltpu.async_copy(rows[b],
                             out_hbm.at[h, pl.ds(cid * NPAD + base, CHUNK)],
                             ssem[b])
        for k in range(nko - 2, nko):
            b = k % 2
            base = sid * RPT + k * CHUNK
            pltpu.make_async_copy(
                rows[b], out_hbm.at[h, pl.ds(cid * NPAD + base, CHUNK)],
                ssem[b]).wait()


_BLK = 1000


def _mm1_body(x_ref, w_ref, degt_ref, h1s_ref, dinv_ref):
    deg = degt_ref[:, 0] + degt_ref[:, 1] + 1.0
    dinv = lax.rsqrt(deg)[:, None]
    h = jnp.dot(x_ref[...], w_ref[...], preferred_element_type=jnp.float32)
    hs = h * dinv
    h1s_ref[0] = hs[:, :DH]
    h1s_ref[1] = hs[:, DH:]
    dinv_ref[...] = dinv


def _mid_body(acc_ref, h1s_ref, dinv_ref, b1_ref, w2_ref, h2s_ref):
    tot = jnp.concatenate(
        [acc_ref[0, 0] + acc_ref[0, 1] + h1s_ref[0],
         acc_ref[1, 0] + acc_ref[1, 1] + h1s_ref[1]], axis=-1)
    dinv = dinv_ref[...]
    h1 = jnp.maximum(tot * dinv + b1_ref[...], 0.0)
    h2s = jnp.dot(h1, w2_ref[...], preferred_element_type=jnp.float32) * dinv
    h2s_ref[0] = h2s[:, :DH]
    h2s_ref[1] = h2s[:, DH:]


def _fin_body(acc_ref, h2s_ref, dinv_ref, b2_ref, out_ref):
    tot = jnp.concatenate(
        [acc_ref[0, 0] + acc_ref[0, 1] + h2s_ref[0],
         acc_ref[1, 0] + acc_ref[1, 1] + h2s_ref[1]], axis=-1)
    out_ref[...] = tot * dinv_ref[...] + b2_ref[...]


def kernel(x, edge_index, W1, b1, W2, b2):
    src = edge_index[0]
    dst = edge_index[1]
    pad = EPAD - E
    src_p = jnp.concatenate([src, jnp.zeros((pad,), jnp.int32)]).reshape(NW, NCHUNK, CHUNK)
    dst_p = jnp.concatenate([dst, jnp.full((pad,), N, jnp.int32)]).reshape(NW, NCHUNK, CHUNK)

    degt = _deg_kernel(dst_p).reshape(NC, NPAD).T

    grid = (N // _BLK,)
    h1s, dinv = pl.pallas_call(
        _mm1_body,
        grid=grid,
        in_specs=[
            pl.BlockSpec((_BLK, D), lambda i: (i, 0)),
            pl.BlockSpec((D, D), lambda i: (0, 0)),
            pl.BlockSpec((_BLK, NC), lambda i: (i, 0)),
        ],
        out_specs=[
            pl.BlockSpec((2, _BLK, DH), lambda i: (0, i, 0)),
            pl.BlockSpec((_BLK, 1), lambda i: (i, 0)),
        ],
        out_shape=[
            jax.ShapeDtypeStruct((2, NPAD, DH), jnp.float32),
            jax.ShapeDtypeStruct((N, 1), jnp.float32),
        ],
    )(x, W1, degt)

    acc1 = _scatter_kernel(src_p, dst_p, h1s).reshape(2, NC, NPAD, DH)

    h2s = pl.pallas_call(
        _mid_body,
        grid=grid,
        in_specs=[
            pl.BlockSpec((2, NC, _BLK, DH), lambda i: (0, 0, i, 0)),
            pl.BlockSpec((2, _BLK, DH), lambda i: (0, i, 0)),
            pl.BlockSpec((_BLK, 1), lambda i: (i, 0)),
            pl.BlockSpec((1, D), lambda i: (0, 0)),
            pl.BlockSpec((D, D), lambda i: (0, 0)),
        ],
        out_specs=pl.BlockSpec((2, _BLK, DH), lambda i: (0, i, 0)),
        out_shape=jax.ShapeDtypeStruct((2, NPAD, DH), jnp.float32),
    )(acc1, h1s, dinv, b1.reshape(1, D), W2)

    acc2 = _scatter_kernel(src_p, dst_p, h2s).reshape(2, NC, NPAD, DH)

    out = pl.pallas_call(
        _fin_body,
        grid=grid,
        in_specs=[
            pl.BlockSpec((2, NC, _BLK, DH), lambda i: (0, 0, i, 0)),
            pl.BlockSpec((2, _BLK, DH), lambda i: (0, i, 0)),
            pl.BlockSpec((_BLK, 1), lambda i: (i, 0)),
            pl.BlockSpec((1, D), lambda i: (0, 0)),
        ],
        out_specs=pl.BlockSpec((_BLK, D), lambda i: (i, 0)),
        out_shape=jax.ShapeDtypeStruct((N, D), jnp.float32),
    )(acc2, h2s, dinv, b2.reshape(1, D))

    return out


# speedup vs baseline: 1.0505x; 1.0015x over previous
import functools

import jax
import jax.numpy as jnp
from jax import lax
from jax.experimental import pallas as pl
from jax.experimental.pallas import tpu as pltpu
from jax.experimental.pallas import tpu_sc as plsc

N = 10000
E = 320000
D = 128

NC = 2
NS = 16
NW = NC * NS

NPAD = 10240
RPT = NPAD // NS
CHUNK = 128
NCHUNK = 80
EPT = CHUNK * NCHUNK
EPAD = NW * EPT
DH = D // 2
IBLK = 8

_MESH = plsc.VectorSubcoreMesh(core_axis_name="c", subcore_axis_name="s")


@functools.partial(
    pl.kernel,
    out_type=jax.ShapeDtypeStruct((NC * NPAD,), jnp.float32),
    mesh=_MESH,
    scratch_types=[
        pltpu.VMEM((NCHUNK, CHUNK), jnp.int32),
        pltpu.VMEM((CHUNK,), jnp.float32),
        pltpu.VMEM((RPT,), jnp.float32),
        pltpu.VMEM_SHARED((NPAD,), jnp.float32),
        pltpu.SemaphoreType.DMA,
    ],
)
def _deg_kernel(dst_hbm, deg_out, idx_v, ones_v, line_v, acc, dsem):
    cid = lax.axis_index("c")
    sid = lax.axis_index("s")
    wid = cid * NS + sid
    z16 = jnp.zeros((16,), jnp.float32)
    for i in range(RPT // 16):
        line_v[pl.ds(i * 16, 16)] = z16
    for i in range(CHUNK // 16):
        ones_v[pl.ds(i * 16, 16)] = jnp.ones((16,), jnp.float32)
    pltpu.sync_copy(line_v, acc.at[pl.ds(sid * RPT, RPT)])
    plsc.subcore_barrier()
    pltpu.sync_copy(dst_hbm.at[wid], idx_v)

    def body(g, carry):
        for b in range(2):
            pltpu.async_copy(ones_v, acc.at[idx_v.at[2 * g + b]], dsem,
                             add=True)
        for b in range(2):
            pltpu.make_async_copy(ones_v, acc.at[idx_v.at[2 * g + b]],
                                  dsem).wait()
        return carry

    lax.fori_loop(0, NCHUNK // 2, body, 0)
    plsc.subcore_barrier()
    pltpu.sync_copy(acc.at[pl.ds(sid * RPT, RPT)], line_v)
    pltpu.sync_copy(line_v, deg_out.at[pl.ds(cid * NPAD + sid * RPT, RPT)])


@functools.partial(
    pl.kernel,
    out_type=jax.ShapeDtypeStruct((2, NC * NPAD, DH), jnp.float32),
    mesh=_MESH,
    compiler_params=pltpu.CompilerParams(use_tc_tiling_on_sc=False),
    scratch_types=[
        pltpu.VMEM((2 * IBLK, CHUNK), jnp.int32),
        pltpu.VMEM((2 * IBLK, CHUNK), jnp.int32),
        pltpu.VMEM((CHUNK, DH), jnp.float32),
        pltpu.VMEM((CHUNK, DH), jnp.float32),
        pltpu.VMEM_SHARED((NPAD, DH), jnp.float32),
        pltpu.VMEM_SHARED((NPAD, DH), jnp.float32),
        pltpu.SemaphoreType.DMA,
        pltpu.SemaphoreType.DMA,
        pltpu.SemaphoreType.DMA,
        pltpu.SemaphoreType.DMA,
        pltpu.SemaphoreType.DMA,
    ],
)
def _scatter_kernel(src_hbm, dst_hbm, tab_hbm, out_hbm, src_v, dst_v,
                    rows0, rows1, tabs, acch, gsem0, gsem1, ssem0, ssem1,
                    isem):
    cid = lax.axis_index("c")
    sid = lax.axis_index("s")
    wid = cid * NS + sid
    rows = (rows0, rows1)
    gsem = (gsem0, gsem1)
    ssem = (ssem0, ssem1)
    z16 = jnp.zeros((16,), jnp.float32)

    for h in range(2):
        for r in range(CHUNK):
            for c in range(DH // 16):
                rows0[r, pl.ds(c * 16, 16)] = z16
        for k in range(RPT // CHUNK):
            base = sid * RPT + k * CHUNK
            pltpu.async_copy(rows0, acch.at[pl.ds(base, CHUNK)], ssem0)
            pltpu.async_copy(tab_hbm.at[h, pl.ds(base, CHUNK)],
                             tabs.at[pl.ds(base, CHUNK)], ssem1)
        for k in range(RPT // CHUNK):
            base = sid * RPT + k * CHUNK
            pltpu.make_async_copy(rows0, acch.at[pl.ds(base, CHUNK)],
                                  ssem0).wait()
            pltpu.make_async_copy(tab_hbm.at[h, pl.ds(base, CHUNK)],
                                  tabs.at[pl.ds(base, CHUNK)], ssem1).wait()
        plsc.subcore_barrier()
        pltpu.sync_copy(src_hbm.at[wid, pl.ds(0, IBLK)],
                        src_v.at[pl.ds(0, IBLK)])
        pltpu.sync_copy(dst_hbm.at[wid, pl.ds(0, IBLK)],
                        dst_v.at[pl.ds(0, IBLK)])
        pltpu.async_copy(tabs.at[src_v.at[0]], rows0, gsem0)

        def body(i, carry):
            for b in range(2):
                j = 2 * i + b
                jm = j % (2 * IBLK)
                jn = (j + 1) % (2 * IBLK)
                jp = (j - 1) % (2 * IBLK)
                pltpu.make_async_copy(tabs.at[src_v.at[jm]], rows[b],
                                      gsem[b]).wait()
                if b == 0:
                    @pl.when(i > 0)
                    def _():
                        pltpu.make_async_copy(rows[1], acch.at[dst_v.at[jp]],
                                              ssem[1]).wait()

                    @pl.when(jnp.logical_and(i % (IBLK // 2) == 0,
                                             j + IBLK < NCHUNK))
                    def _():
                        nxt = pl.multiple_of(j + IBLK, IBLK)
                        off = pl.multiple_of((jm + IBLK) % (2 * IBLK), IBLK)
                        pltpu.async_copy(src_hbm.at[wid, pl.ds(nxt, IBLK)],
                                         src_v.at[pl.ds(off, IBLK)], isem)
                        pltpu.async_copy(dst_hbm.at[wid, pl.ds(nxt, IBLK)],
                                         dst_v.at[pl.ds(off, IBLK)], isem)
                    pltpu.async_copy(tabs.at[src_v.at[jn]], rows[1], gsem[1])
                else:
                    pltpu.make_async_copy(rows[0], acch.at[dst_v.at[jp]],
                                          ssem[0]).wait()

                    @pl.when(jnp.logical_and(i % (IBLK // 2) == IBLK // 2 - 1,
                                             j + 1 < NCHUNK))
                    def _():
                        nxt = pl.multiple_of(j + 1, IBLK)
                        off = pl.multiple_of((j + 1) % (2 * IBLK), IBLK)
                        pltpu.make_async_copy(
                            src_hbm.at[wid, pl.ds(nxt, IBLK)],
                            src_v.at[pl.ds(off, IBLK)], isem).wait()
                        pltpu.make_async_copy(
                            dst_hbm.at[wid, pl.ds(nxt, IBLK)],
                            dst_v.at[pl.ds(off, IBLK)], isem).wait()

                    @pl.when(i < NCHUNK // 2 - 1)
                    def _():
                        pltpu.async_copy(tabs.at[src_v.at[jn]], rows[0],
                                         gsem[0])
                pltpu.async_copy(rows[b], acch.at[dst_v.at[jm]], ssem[b],
                                 add=True)
            return carry

        lax.fori_loop(0, NCHUNK // 2, body, 0)
        pltpu.make_async_copy(
            rows1, acch.at[dst_v.at[(NCHUNK - 1) % (2 * IBLK)]], ssem1).wait()
        plsc.subcore_barrier()
        nko = RPT // CHUNK
        for k in range(nko):
            b = k % 2
            base = sid * RPT + k * CHUNK
            if k >= 2:
                pb = sid * RPT + (k - 2) * CHUNK
                pltpu.make_async_copy(
                    rows[b], out_hbm.at[h, pl.ds(cid * NPAD + pb, CHUNK)],
                    ssem[b]).wait()
            pltpu.async_copy(acch.at[pl.ds(base, CHUNK)], rows[b], gsem[b])
            pltpu.make_async_copy(acch.at[pl.ds(base, CHUNK)], rows[b],
                                  gsem[b]).wait()
            pltpu.async_copy(rows[b],
                             out_hbm.at[h, pl.ds(cid * NPAD + base, CHUNK)],
                             ssem[b])
        for k in range(nko - 2, nko):
            b = k % 2
            base = sid * RPT + k * CHUNK
            pltpu.make_async_copy(
                rows[b], out_hbm.at[h, pl.ds(cid * NPAD + base, CHUNK)],
                ssem[b]).wait()


_BLK = 1000


def _mm1_body(x_ref, w_ref, degt_ref, h1s_ref, dinv_ref):
    deg = degt_ref[:, 0] + degt_ref[:, 1] + 1.0
    dinv = lax.rsqrt(deg)[:, None]
    h = jnp.dot(x_ref[...], w_ref[...], preferred_element_type=jnp.float32)
    hs = h * dinv
    h1s_ref[0] = hs[:, :DH]
    h1s_ref[1] = hs[:, DH:]
    dinv_ref[...] = dinv


def _mid_body(acc_ref, h1s_ref, dinv_ref, b1_ref, w2_ref, h2s_ref):
    tot = jnp.concatenate(
        [acc_ref[0, 0] + acc_ref[0, 1] + h1s_ref[0],
         acc_ref[1, 0] + acc_ref[1, 1] + h1s_ref[1]], axis=-1)
    dinv = dinv_ref[...]
    h1 = jnp.maximum(tot * dinv + b1_ref[...], 0.0)
    h2s = jnp.dot(h1, w2_ref[...], preferred_element_type=jnp.float32) * dinv
    h2s_ref[0] = h2s[:, :DH]
    h2s_ref[1] = h2s[:, DH:]


def _fin_body(acc_ref, h2s_ref, dinv_ref, b2_ref, out_ref):
    tot = jnp.concatenate(
        [acc_ref[0, 0] + acc_ref[0, 1] + h2s_ref[0],
         acc_ref[1, 0] + acc_ref[1, 1] + h2s_ref[1]], axis=-1)
    out_ref[...] = tot * dinv_ref[...] + b2_ref[...]


def kernel(x, edge_index, W1, b1, W2, b2):
    src = edge_index[0]
    dst = edge_index[1]
    pad = EPAD - E
    src_p = jnp.concatenate([src, jnp.zeros((pad,), jnp.int32)]).reshape(NW, NCHUNK, CHUNK)
    dst_p = jnp.concatenate([dst, jnp.full((pad,), N, jnp.int32)]).reshape(NW, NCHUNK, CHUNK)

    degt = _deg_kernel(dst_p).reshape(NC, NPAD).T

    grid = (N // _BLK,)
    h1s, dinv = pl.pallas_call(
        _mm1_body,
        grid=grid,
        in_specs=[
            pl.BlockSpec((_BLK, D), lambda i: (i, 0)),
            pl.BlockSpec((D, D), lambda i: (0, 0)),
            pl.BlockSpec((_BLK, NC), lambda i: (i, 0)),
        ],
        out_specs=[
            pl.BlockSpec((2, _BLK, DH), lambda i: (0, i, 0)),
            pl.BlockSpec((_BLK, 1), lambda i: (i, 0)),
        ],
        out_shape=[
            jax.ShapeDtypeStruct((2, NPAD, DH), jnp.float32),
            jax.ShapeDtypeStruct((N, 1), jnp.float32),
        ],
    )(x, W1, degt)

    acc1 = _scatter_kernel(src_p, dst_p, h1s).reshape(2, NC, NPAD, DH)

    h2s = pl.pallas_call(
        _mid_body,
        grid=grid,
        in_specs=[
            pl.BlockSpec((2, NC, _BLK, DH), lambda i: (0, 0, i, 0)),
            pl.BlockSpec((2, _BLK, DH), lambda i: (0, i, 0)),
            pl.BlockSpec((_BLK, 1), lambda i: (i, 0)),
            pl.BlockSpec((1, D), lambda i: (0, 0)),
            pl.BlockSpec((D, D), lambda i: (0, 0)),
        ],
        out_specs=pl.BlockSpec((2, _BLK, DH), lambda i: (0, i, 0)),
        out_shape=jax.ShapeDtypeStruct((2, NPAD, DH), jnp.float32),
    )(acc1, h1s, dinv, b1.reshape(1, D), W2)

    acc2 = _scatter_kernel(src_p, dst_p, h2s).reshape(2, NC, NPAD, DH)

    out = pl.pallas_call(
        _fin_body,
        grid=grid,
        in_specs=[
            pl.BlockSpec((2, NC, _BLK, DH), lambda i: (0, 0, i, 0)),
            pl.BlockSpec((2, _BLK, DH), lambda i: (0, i, 0)),
            pl.BlockSpec((_BLK, 1), lambda i: (i, 0)),
            pl.BlockSpec((1, D), lambda i: (0, 0)),
        ],
        out_specs=pl.BlockSpec((_BLK, D), lambda i: (i, 0)),
        out_shape=jax.ShapeDtypeStruct((N, D), jnp.float32),
    )(acc2, h2s, dinv, b2.reshape(1, D))

    return out
